# baseline (speedup 1.0000x reference)
import functools

import jax
import jax.numpy as jnp
from jax import lax
from jax.experimental import pallas as pl
from jax.experimental.pallas import tpu as pltpu
from jax.experimental.pallas import tpu_sc as plsc

_B, _S, _D = 1, 2048, 768
_F = 3072
_E = 8
_JITTER = 0.01

_BLK = 256
_NBMAX = 24
_NSLOT = _NBMAX * _BLK
_NW = 32
_TPW = _S // _NW

_NRB = _S // _BLK


def _routing_body(x_ref, gw_ref, logits_ref, mult_ref, slots_ref, meta_ref):
    x = x_ref[...]
    gw = gw_ref[...]
    scores = jnp.dot(x, gw, preferred_element_type=jnp.float32)
    logits_ref[...] = scores

    neg = jnp.float32(-jnp.inf)
    lane = jax.lax.broadcasted_iota(jnp.int32, (_S, _E), 1)

    max_vals = jnp.max(scores, axis=-1, keepdims=True)
    max_ind = jnp.min(jnp.where(scores == max_vals, lane, _E),
                      axis=-1, keepdims=True)
    factor = jnp.maximum(jnp.abs(scores), max_vals)
    mask1 = (max_vals - scores) / factor > 2 * _JITTER
    mg = jnp.where(mask1, neg, scores)
    m = jnp.max(mg, axis=-1, keepdims=True)
    p = jnp.exp(mg - m)
    p = p / jnp.sum(p, axis=-1, keepdims=True)
    onehot1 = lane == max_ind
    mult1 = jnp.sum(jnp.where(onehot1, p, 0.0), axis=-1, keepdims=True)

    masked_scores = jnp.where(onehot1, neg, scores)
    max_vals2 = jnp.max(masked_scores, axis=-1, keepdims=True)
    max_ind2 = jnp.min(jnp.where(masked_scores == max_vals2, lane, _E),
                       axis=-1, keepdims=True)
    factor2 = jnp.maximum(jnp.abs(scores), max_vals2)
    mask2 = (max_vals2 - scores) / factor2 > 2 * _JITTER
    mg2 = jnp.where(mask2, neg, masked_scores)
    m2 = jnp.max(mg2, axis=-1, keepdims=True)
    p2 = jnp.exp(mg2 - m2)
    p2 = p2 / jnp.sum(p2, axis=-1, keepdims=True)
    onehot2 = lane == max_ind2
    mult2 = jnp.sum(jnp.where(onehot2, p2, 0.0), axis=-1, keepdims=True)

    mult_ref[...] = (jnp.where(lane == 0, mult1, 0.0)
                     + jnp.where(lane == 1, mult2, 0.0))

    oh0 = onehot1.astype(jnp.float32)
    oh1 = onehot2.astype(jnp.float32)
    tri = (jax.lax.broadcasted_iota(jnp.int32, (_BLK, _BLK), 1)
           < jax.lax.broadcasted_iota(jnp.int32, (_BLK, _BLK), 0)
           ).astype(jnp.float32)

    def excl_cumsum(oh, carry0):
        blocks = []
        carry = carry0
        for i in range(_NRB):
            blk = oh[i * _BLK:(i + 1) * _BLK, :]
            c = jnp.dot(tri, blk, preferred_element_type=jnp.float32,
                        precision=jax.lax.Precision.HIGHEST) + carry
            blocks.append(c)
            carry = carry + jnp.sum(blk, axis=0, keepdims=True)
        return jnp.concatenate(blocks, axis=0), carry

    zero8 = jnp.zeros((1, _E), jnp.float32)
    c0, tot0 = excl_cumsum(oh0, zero8)
    c1, tot01 = excl_cumsum(oh1, tot0)

    rank0 = jnp.sum(jnp.where(onehot1, c0, 0.0), axis=-1, keepdims=True)
    rank1 = jnp.sum(jnp.where(onehot2, c1, 0.0), axis=-1, keepdims=True)

    counts = tot01.astype(jnp.int32)
    nb = (counts + (_BLK - 1)) // _BLK
    lane8 = jax.lax.broadcasted_iota(jnp.int32, (1, _E), 1)
    nb_s = [jnp.sum(jnp.where(lane8 == e, nb, 0)) for e in range(_E)]
    cb_s = []
    acc = nb_s[0]
    for e in range(_E):
        if e:
            acc = acc + nb_s[e]
        cb_s.append(acc)
    off_s = [(cb_s[e] - nb_s[e]) * _BLK for e in range(_E)]
    off = jnp.zeros((1, _E), jnp.int32)
    for e in range(_E):
        off = jnp.where(lane8 == e, off_s[e], off)

    offf = off.astype(jnp.float32)
    off0 = jnp.sum(jnp.where(onehot1, offf, 0.0), axis=-1, keepdims=True)
    off1 = jnp.sum(jnp.where(onehot2, offf, 0.0), axis=-1, keepdims=True)
    slot0 = (off0 + rank0).astype(jnp.int32)
    slot1 = (off1 + rank1).astype(jnp.int32)
    slots_ref[...] = (jnp.where(lane == 0, slot0, 0)
                      + jnp.where(lane == 1, slot1, 0))

    nact = cb_s[_E - 1]
    lane128 = jax.lax.broadcasted_iota(jnp.int32, (1, 128), 1)
    raw = jnp.zeros((1, 128), jnp.int32)
    for e in range(_E):
        raw = raw + (lane128 >= cb_s[e]).astype(jnp.int32)
    lastex = jnp.max(jnp.where(lane128 < nact, raw, 0))
    be = jnp.minimum(raw, lastex)
    xmap = jnp.minimum(lane128 - 32, nact - 1)
    meta = jnp.where(lane128 < _NBMAX, be, 0)
    meta = jnp.where((lane128 >= 32) & (lane128 < 32 + _NBMAX), xmap, meta)
    meta = jnp.where(lane128 == 56, nact, meta)
    meta_ref[...] = meta


def _routing(x, gate_w):
    return pl.pallas_call(
        _routing_body,
        out_shape=[
            jax.ShapeDtypeStruct((_S, _E), jnp.float32),
            jax.ShapeDtypeStruct((_S, _E), jnp.float32),
            jax.ShapeDtypeStruct((_S, _E), jnp.int32),
            jax.ShapeDtypeStruct((1, 128), jnp.int32),
        ],
    )(x, gate_w)


def _gmm_body(meta_ref, xs_ref, w1_ref, w3_ref, w2_ref, ds_ref):
    b = pl.program_id(0)
    nact = meta_ref[56]

    @pl.when(b < nact)
    def _():
        x = xs_ref[...].astype(jnp.bfloat16)
        g = jnp.dot(x, w1_ref[0], preferred_element_type=jnp.float32)
        u = jnp.dot(x, w3_ref[0], preferred_element_type=jnp.float32)
        h = g * (1.0 / (1.0 + jnp.exp(-g))) * u
        ds_ref[...] = jnp.dot(h.astype(jnp.bfloat16), w2_ref[0],
                              preferred_element_type=jnp.float32)


def _gmm(meta128, xs, w1, w3, w2):
    grid_spec = pltpu.PrefetchScalarGridSpec(
        num_scalar_prefetch=1,
        grid=(_NBMAX,),
        in_specs=[
            pl.BlockSpec((_BLK, _D), lambda b, m: (m[32 + b], 0)),
            pl.BlockSpec((1, _D, _F), lambda b, m: (m[b], 0, 0)),
            pl.BlockSpec((1, _D, _F), lambda b, m: (m[b], 0, 0)),
            pl.BlockSpec((1, _F, _D), lambda b, m: (m[b], 0, 0)),
        ],
        out_specs=pl.BlockSpec((_BLK, _D), lambda b, m: (b, 0)),
    )
    return pl.pallas_call(
        _gmm_body,
        grid_spec=grid_spec,
        out_shape=jax.ShapeDtypeStruct((_NSLOT, _D), jnp.float32),
        compiler_params=pltpu.CompilerParams(
            dimension_semantics=("arbitrary",),
        ),
    )(meta128, xs, w1, w3, w2)


def _dispatch(x, s0, s1):
    mesh = plsc.VectorSubcoreMesh(core_axis_name="c", subcore_axis_name="s")

    @functools.partial(
        pl.kernel, mesh=mesh,
        out_type=jax.ShapeDtypeStruct((_NSLOT, _D), jnp.float32),
        scratch_types=[
            pltpu.VMEM((_TPW, _D), jnp.float32),
            pltpu.VMEM((_TPW,), jnp.int32),
            pltpu.VMEM((_TPW,), jnp.int32),
            pltpu.SemaphoreType.DMA,
            pltpu.SemaphoreType.DMA,
        ],
    )
    def k(x_hbm, s0_hbm, s1_hbm, out_hbm, xbuf, i0, i1, sem0, sem1):
        wid = lax.axis_index("s") * 2 + lax.axis_index("c")
        base = wid * _TPW
        pltpu.sync_copy(x_hbm.at[pl.ds(base, _TPW)], xbuf)
        pltpu.sync_copy(s0_hbm.at[wid], i0)
        pltpu.sync_copy(s1_hbm.at[wid], i1)
        c0 = pltpu.async_copy(xbuf, out_hbm.at[i0], sem0)
        c1 = pltpu.async_copy(xbuf, out_hbm.at[i1], sem1)
        c0.wait()
        c1.wait()

    return k(x, s0, s1)


def _combine(ds, s0, s1):
    mesh = plsc.VectorSubcoreMesh(core_axis_name="c", subcore_axis_name="s")

    @functools.partial(
        pl.kernel, mesh=mesh,
        out_type=[
            jax.ShapeDtypeStruct((_S, _D), jnp.float32),
            jax.ShapeDtypeStruct((_S, _D), jnp.float32),
        ],
        scratch_types=[
            pltpu.VMEM((_TPW, _D), jnp.float32),
            pltpu.VMEM((_TPW, _D), jnp.float32),
            pltpu.VMEM((_TPW,), jnp.int32),
            pltpu.VMEM((_TPW,), jnp.int32),
            pltpu.SemaphoreType.DMA,
            pltpu.SemaphoreType.DMA,
        ],
    )
    def k(ds_hbm, s0_hbm, s1_hbm, d0_hbm, d1_hbm,
          buf0, buf1, i0, i1, sem0, sem1):
        wid = lax.axis_index("s") * 2 + lax.axis_index("c")
        base = wid * _TPW
        pltpu.sync_copy(s0_hbm.at[wid], i0)
        pltpu.sync_copy(s1_hbm.at[wid], i1)
        c0 = pltpu.async_copy(ds_hbm.at[i0], buf0, sem0)
        c1 = pltpu.async_copy(ds_hbm.at[i1], buf1, sem1)
        c0.wait()
        c1.wait()
        pltpu.sync_copy(buf0, d0_hbm.at[pl.ds(base, _TPW)])
        pltpu.sync_copy(buf1, d1_hbm.at[pl.ds(base, _TPW)])

    return k(ds, s0, s1)


def _wadd_body(d0_ref, d1_ref, m_ref, out_ref):
    m = m_ref[...]
    out_ref[...] = d0_ref[...] * m[:, 0:1] + d1_ref[...] * m[:, 1:2]


def _wadd(d0, d1, mult):
    return pl.pallas_call(
        _wadd_body,
        grid=(_NRB,),
        in_specs=[
            pl.BlockSpec((_BLK, _D), lambda rb: (rb, 0)),
            pl.BlockSpec((_BLK, _D), lambda rb: (rb, 0)),
            pl.BlockSpec((_BLK, _E), lambda rb: (rb, 0)),
        ],
        out_specs=pl.BlockSpec((_BLK, _D), lambda rb: (rb, 0)),
        out_shape=jax.ShapeDtypeStruct((_S, _D), jnp.float32),
    )(d0, d1, mult)


def kernel(hidden_states, gate_w, w1, w2, w3):
    x = hidden_states.reshape(-1, _D)
    logits, mult, slots, meta = _routing(x, gate_w)
    meta128 = meta.reshape(128)
    s0 = slots[:, 0].reshape(_NW, _TPW)
    s1 = slots[:, 1].reshape(_NW, _TPW)
    xs = _dispatch(x, s0, s1)
    ds = _gmm(meta128, xs,
              w1.astype(jnp.bfloat16),
              w3.astype(jnp.bfloat16),
              w2.astype(jnp.bfloat16))
    d0, d1 = _combine(ds, s0, s1)
    out = _wadd(d0, d1, mult)
    return (out.reshape(hidden_states.shape),
            logits.reshape(_B, _S, _E))


# speedup vs baseline: 1.6205x; 1.6205x over previous
import functools

import jax
import jax.numpy as jnp
from jax import lax
from jax.experimental import pallas as pl
from jax.experimental.pallas import tpu as pltpu
from jax.experimental.pallas import tpu_sc as plsc

_B, _S, _D = 1, 2048, 768
_F = 3072
_E = 8
_JITTER = 0.01

_BLK = 256
_NBMAX = 24
_NSLOT = _NBMAX * _BLK
_NW = 32
_TPW = _S // _NW

_NRB = _S // _BLK


def _routing_body(x_ref, gw_ref, logits_ref, mult_ref, slots_ref, meta_ref):
    x = x_ref[...]
    gw = gw_ref[...]
    scores = jnp.dot(x, gw, preferred_element_type=jnp.float32)
    logits_ref[...] = scores

    neg = jnp.float32(-jnp.inf)
    lane = jax.lax.broadcasted_iota(jnp.int32, (_S, _E), 1)

    max_vals = jnp.max(scores, axis=-1, keepdims=True)
    max_ind = jnp.min(jnp.where(scores == max_vals, lane, _E),
                      axis=-1, keepdims=True)
    factor = jnp.maximum(jnp.abs(scores), max_vals)
    mask1 = (max_vals - scores) / factor > 2 * _JITTER
    mg = jnp.where(mask1, neg, scores)
    m = jnp.max(mg, axis=-1, keepdims=True)
    p = jnp.exp(mg - m)
    p = p / jnp.sum(p, axis=-1, keepdims=True)
    onehot1 = lane == max_ind
    mult1 = jnp.sum(jnp.where(onehot1, p, 0.0), axis=-1, keepdims=True)

    masked_scores = jnp.where(onehot1, neg, scores)
    max_vals2 = jnp.max(masked_scores, axis=-1, keepdims=True)
    max_ind2 = jnp.min(jnp.where(masked_scores == max_vals2, lane, _E),
                       axis=-1, keepdims=True)
    factor2 = jnp.maximum(jnp.abs(scores), max_vals2)
    mask2 = (max_vals2 - scores) / factor2 > 2 * _JITTER
    mg2 = jnp.where(mask2, neg, masked_scores)
    m2 = jnp.max(mg2, axis=-1, keepdims=True)
    p2 = jnp.exp(mg2 - m2)
    p2 = p2 / jnp.sum(p2, axis=-1, keepdims=True)
    onehot2 = lane == max_ind2
    mult2 = jnp.sum(jnp.where(onehot2, p2, 0.0), axis=-1, keepdims=True)

    mult_ref[...] = jnp.concatenate(
        [jnp.broadcast_to(mult1, (_S, 16)),
         jnp.broadcast_to(mult2, (_S, 16))], axis=-1)

    oh0 = onehot1.astype(jnp.float32)
    oh1 = onehot2.astype(jnp.float32)
    tri = (jax.lax.broadcasted_iota(jnp.int32, (_BLK, _BLK), 1)
           < jax.lax.broadcasted_iota(jnp.int32, (_BLK, _BLK), 0)
           ).astype(jnp.float32)

    def excl_cumsum(oh, carry0):
        blocks = []
        carry = carry0
        for i in range(_NRB):
            blk = oh[i * _BLK:(i + 1) * _BLK, :]
            c = jnp.dot(tri, blk, preferred_element_type=jnp.float32,
                        precision=jax.lax.Precision.HIGHEST) + carry
            blocks.append(c)
            carry = carry + jnp.sum(blk, axis=0, keepdims=True)
        return jnp.concatenate(blocks, axis=0), carry

    zero8 = jnp.zeros((1, _E), jnp.float32)
    c0, tot0 = excl_cumsum(oh0, zero8)
    c1, tot01 = excl_cumsum(oh1, tot0)

    rank0 = jnp.sum(jnp.where(onehot1, c0, 0.0), axis=-1, keepdims=True)
    rank1 = jnp.sum(jnp.where(onehot2, c1, 0.0), axis=-1, keepdims=True)

    counts = tot01.astype(jnp.int32)
    nb = (counts + (_BLK - 1)) // _BLK
    lane8 = jax.lax.broadcasted_iota(jnp.int32, (1, _E), 1)
    nb_s = [jnp.sum(jnp.where(lane8 == e, nb, 0)) for e in range(_E)]
    cb_s = []
    acc = nb_s[0]
    for e in range(_E):
        if e:
            acc = acc + nb_s[e]
        cb_s.append(acc)
    off_s = [(cb_s[e] - nb_s[e]) * _BLK for e in range(_E)]
    off = jnp.zeros((1, _E), jnp.int32)
    for e in range(_E):
        off = jnp.where(lane8 == e, off_s[e], off)

    offf = off.astype(jnp.float32)
    off0 = jnp.sum(jnp.where(onehot1, offf, 0.0), axis=-1, keepdims=True)
    off1 = jnp.sum(jnp.where(onehot2, offf, 0.0), axis=-1, keepdims=True)
    slot0 = (off0 + rank0).astype(jnp.int32)
    slot1 = (off1 + rank1).astype(jnp.int32)
    slots_ref[...] = (jnp.where(lane == 0, slot0, 0)
                      + jnp.where(lane == 1, slot1, 0))

    nact = cb_s[_E - 1]
    lane128 = jax.lax.broadcasted_iota(jnp.int32, (1, 128), 1)

    has = [(nb_s[e] > 0).astype(jnp.int32) for e in range(_E)]
    rank = []
    racc = jnp.zeros((), jnp.int32)
    for e in range(_E):
        rank.append(racc)
        racc = racc + has[e]
    nruns = racc

    def block_tables(brow):
        raw = jnp.zeros(brow.shape, jnp.int32)
        for e in range(_E):
            raw = raw + (brow >= cb_s[e]).astype(jnp.int32)
        return raw

    raw0 = block_tables(lane128)
    lastex = jnp.max(jnp.where(lane128 < nact, raw0, 0))
    be = jnp.minimum(raw0, lastex)
    xmap = jnp.minimum(lane128 - 32, nact - 1)

    bew_rid = jnp.minimum(block_tables(lane128 - 80), lastex)
    rid = jnp.zeros((1, 128), jnp.int32)
    for e in range(_E):
        rid = jnp.where(bew_rid == e, rank[e], rid)

    bew_st = jnp.minimum(block_tables(lane128 - 104), lastex)
    st = jnp.zeros((1, 128), jnp.int32)
    for e in range(_E):
        st = jnp.where(
            jnp.logical_and(bew_st == e,
                            (lane128 - 104) == cb_s[e] - nb_s[e]),
            1, st)
    st = jnp.where(lane128 - 104 < nact, st, 0)

    re = jnp.zeros((1, 128), jnp.int32)
    for e in range(_E):
        re = jnp.where(
            jnp.logical_and(lane128 == 64 + rank[e], has[e] > 0), e, re)

    meta = jnp.where(lane128 < _NBMAX, be, 0)
    meta = jnp.where((lane128 >= 32) & (lane128 < 32 + _NBMAX), xmap, meta)
    meta = jnp.where(lane128 == 56, nact, meta)
    meta = jnp.where((lane128 >= 64) & (lane128 < 64 + _E), re, meta)
    meta = jnp.where(lane128 == 72, nruns, meta)
    meta = jnp.where((lane128 >= 80) & (lane128 < 80 + _NBMAX), rid, meta)
    meta = jnp.where(lane128 >= 104, st, meta)
    meta_ref[...] = meta


def _routing(x, gate_w):
    return pl.pallas_call(
        _routing_body,
        out_shape=[
            jax.ShapeDtypeStruct((_S, _E), jnp.float32),
            jax.ShapeDtypeStruct((_S, 32), jnp.float32),
            jax.ShapeDtypeStruct((_S, _E), jnp.int32),
            jax.ShapeDtypeStruct((1, 128), jnp.int32),
        ],
    )(x, gate_w)


def _w_copies(w1_hbm, w3_hbm, w2_hbm, vw1, vw3, vw2, sem, e, slot):
    return (
        pltpu.make_async_copy(w1_hbm.at[e], vw1.at[slot], sem.at[slot]),
        pltpu.make_async_copy(w3_hbm.at[e], vw3.at[slot], sem.at[slot]),
        pltpu.make_async_copy(w2_hbm.at[e], vw2.at[slot], sem.at[slot]),
    )


def _gmm_body(meta_ref, xs_ref, w1_hbm, w3_hbm, w2_hbm, ds_ref,
              vw1, vw3, vw2, sem):
    b = pl.program_id(0)
    nact = meta_ref[56]
    nruns = meta_ref[72]
    rid = meta_ref[80 + b]
    is_start = meta_ref[104 + b]
    slot = jax.lax.rem(rid, 2)

    @pl.when(b == 0)
    def _():
        for c in _w_copies(w1_hbm, w3_hbm, w2_hbm, vw1, vw3, vw2, sem,
                           meta_ref[64], 0):
            c.start()

        @pl.when(nruns > 1)
        def _():
            for c in _w_copies(w1_hbm, w3_hbm, w2_hbm, vw1, vw3, vw2, sem,
                               meta_ref[65], 1):
                c.start()

    @pl.when(jnp.logical_and(b < nact, is_start == 1))
    def _():
        for c in _w_copies(w1_hbm, w3_hbm, w2_hbm, vw1, vw3, vw2, sem,
                           meta_ref[64 + rid], slot):
            c.wait()

        @pl.when(jnp.logical_and(rid + 1 < nruns, rid >= 1))
        def _():
            for c in _w_copies(w1_hbm, w3_hbm, w2_hbm, vw1, vw3, vw2, sem,
                               meta_ref[64 + rid + 1], 1 - slot):
                c.start()

    @pl.when(b < nact)
    def _():
        x = xs_ref[...]
        g = jnp.dot(x, vw1[slot], preferred_element_type=jnp.float32)
        u = jnp.dot(x, vw3[slot], preferred_element_type=jnp.float32)
        h = g * (1.0 / (1.0 + jnp.exp(-g))) * u
        ds_ref[...] = jnp.dot(h, vw2[slot], preferred_element_type=jnp.float32)


def _gmm(meta128, xs, w1, w3, w2):
    grid_spec = pltpu.PrefetchScalarGridSpec(
        num_scalar_prefetch=1,
        grid=(_NBMAX,),
        in_specs=[
            pl.BlockSpec((_BLK, _D), lambda b, m: (m[32 + b], 0)),
            pl.BlockSpec(memory_space=pl.ANY),
            pl.BlockSpec(memory_space=pl.ANY),
            pl.BlockSpec(memory_space=pl.ANY),
        ],
        out_specs=pl.BlockSpec((_BLK, _D), lambda b, m: (b, 0)),
        scratch_shapes=[
            pltpu.VMEM((2, _D, _F), jnp.float32),
            pltpu.VMEM((2, _D, _F), jnp.float32),
            pltpu.VMEM((2, _F, _D), jnp.float32),
            pltpu.SemaphoreType.DMA((2,)),
        ],
    )
    return pl.pallas_call(
        _gmm_body,
        grid_spec=grid_spec,
        out_shape=jax.ShapeDtypeStruct((_NSLOT, _D), jnp.float32),
        compiler_params=pltpu.CompilerParams(
            dimension_semantics=("arbitrary",),
            vmem_limit_bytes=100 * 1024 * 1024,
        ),
    )(meta128, xs, w1, w3, w2)


def _dispatch(x, s0, s1):
    mesh = plsc.VectorSubcoreMesh(core_axis_name="c", subcore_axis_name="s")

    @functools.partial(
        pl.kernel, mesh=mesh,
        out_type=jax.ShapeDtypeStruct((_NSLOT, _D), jnp.float32),
        scratch_types=[
            pltpu.VMEM((_TPW, _D), jnp.float32),
            pltpu.VMEM((_TPW,), jnp.int32),
            pltpu.VMEM((_TPW,), jnp.int32),
            pltpu.SemaphoreType.DMA,
            pltpu.SemaphoreType.DMA,
        ],
    )
    def k(x_hbm, s0_hbm, s1_hbm, out_hbm, xbuf, i0, i1, sem0, sem1):
        wid = lax.axis_index("s") * 2 + lax.axis_index("c")
        base = wid * _TPW
        pltpu.sync_copy(x_hbm.at[pl.ds(base, _TPW)], xbuf)
        pltpu.sync_copy(s0_hbm.at[wid], i0)
        pltpu.sync_copy(s1_hbm.at[wid], i1)
        c0 = pltpu.async_copy(xbuf, out_hbm.at[i0], sem0)
        c1 = pltpu.async_copy(xbuf, out_hbm.at[i1], sem1)
        c0.wait()
        c1.wait()

    return k(x, s0, s1)


def _combine(ds, s0, s1, mult):
    mesh = plsc.VectorSubcoreMesh(core_axis_name="c", subcore_axis_name="s")

    @functools.partial(
        pl.kernel, mesh=mesh,
        out_type=jax.ShapeDtypeStruct((_S, _D), jnp.float32),
        scratch_types=[
            pltpu.VMEM((_TPW, _D), jnp.float32),
            pltpu.VMEM((_TPW, _D), jnp.float32),
            pltpu.VMEM((_TPW, 32), jnp.float32),
            pltpu.VMEM((_TPW,), jnp.int32),
            pltpu.VMEM((_TPW,), jnp.int32),
            pltpu.SemaphoreType.DMA,
            pltpu.SemaphoreType.DMA,
        ],
    )
    def k(ds_hbm, s0_hbm, s1_hbm, m_hbm, out_hbm,
          buf0, buf1, mbuf, i0, i1, sem0, sem1):
        wid = lax.axis_index("s") * 2 + lax.axis_index("c")
        base = wid * _TPW
        pltpu.sync_copy(s0_hbm.at[wid], i0)
        pltpu.sync_copy(s1_hbm.at[wid], i1)
        pltpu.sync_copy(m_hbm.at[pl.ds(base, _TPW)], mbuf)
        c0 = pltpu.async_copy(ds_hbm.at[i0], buf0, sem0)
        c1 = pltpu.async_copy(ds_hbm.at[i1], buf1, sem1)
        c0.wait()
        c1.wait()

        def row(j, _):
            w0 = mbuf[j, pl.ds(0, 16)]
            w1 = mbuf[j, pl.ds(16, 16)]
            for c in range(_D // 16):
                cs = pl.ds(c * 16, 16)
                buf0[j, cs] = buf0[j, cs] * w0 + buf1[j, cs] * w1
            return ()

        lax.fori_loop(0, _TPW, row, ())
        pltpu.sync_copy(buf0, out_hbm.at[pl.ds(base, _TPW)])

    return k(ds, s0, s1, mult)


def kernel(hidden_states, gate_w, w1, w2, w3):
    x = hidden_states.reshape(-1, _D)
    logits, mult, slots, meta = _routing(x, gate_w)
    meta128 = meta.reshape(128)
    s0 = slots[:, 0].reshape(_NW, _TPW)
    s1 = slots[:, 1].reshape(_NW, _TPW)
    xs = _dispatch(x, s0, s1)
    ds = _gmm(meta128, xs, w1, w3, w2)
    out = _combine(ds, s0, s1, mult)
    return (out.reshape(hidden_states.shape),
            logits.reshape(_B, _S, _E))


# speedup vs baseline: 1.6272x; 1.0041x over previous
import functools

import jax
import jax.numpy as jnp
from jax import lax
from jax.experimental import pallas as pl
from jax.experimental.pallas import tpu as pltpu
from jax.experimental.pallas import tpu_sc as plsc

_B, _S, _D = 1, 2048, 768
_F = 3072
_E = 8
_JITTER = 0.01

_BLK = 256
_NBMAX = 24
_NSLOT = _NBMAX * _BLK
_NW = 32
_TPW = _S // _NW

_NRB = _S // _BLK


def _routing_body(x_ref, gw_ref, logits_ref, mult_ref, slots_ref, meta_ref):
    x = x_ref[...]
    gw = gw_ref[...]
    scores = jnp.dot(x, gw, preferred_element_type=jnp.float32)
    logits_ref[...] = scores

    neg = jnp.float32(-jnp.inf)
    lane = jax.lax.broadcasted_iota(jnp.int32, (_S, _E), 1)

    max_vals = jnp.max(scores, axis=-1, keepdims=True)
    max_ind = jnp.min(jnp.where(scores == max_vals, lane, _E),
                      axis=-1, keepdims=True)
    factor = jnp.maximum(jnp.abs(scores), max_vals)
    mask1 = (max_vals - scores) / factor > 2 * _JITTER
    mg = jnp.where(mask1, neg, scores)
    m = jnp.max(mg, axis=-1, keepdims=True)
    p = jnp.exp(mg - m)
    p = p / jnp.sum(p, axis=-1, keepdims=True)
    onehot1 = lane == max_ind
    mult1 = jnp.sum(jnp.where(onehot1, p, 0.0), axis=-1, keepdims=True)

    masked_scores = jnp.where(onehot1, neg, scores)
    max_vals2 = jnp.max(masked_scores, axis=-1, keepdims=True)
    max_ind2 = jnp.min(jnp.where(masked_scores == max_vals2, lane, _E),
                       axis=-1, keepdims=True)
    factor2 = jnp.maximum(jnp.abs(scores), max_vals2)
    mask2 = (max_vals2 - scores) / factor2 > 2 * _JITTER
    mg2 = jnp.where(mask2, neg, masked_scores)
    m2 = jnp.max(mg2, axis=-1, keepdims=True)
    p2 = jnp.exp(mg2 - m2)
    p2 = p2 / jnp.sum(p2, axis=-1, keepdims=True)
    onehot2 = lane == max_ind2
    mult2 = jnp.sum(jnp.where(onehot2, p2, 0.0), axis=-1, keepdims=True)

    mult_ref[...] = jnp.concatenate(
        [jnp.broadcast_to(mult1, (_S, 16)),
         jnp.broadcast_to(mult2, (_S, 16))], axis=-1)

    oh0 = onehot1.astype(jnp.float32)
    oh1 = onehot2.astype(jnp.float32)
    tri = (jax.lax.broadcasted_iota(jnp.int32, (_BLK, _BLK), 1)
           < jax.lax.broadcasted_iota(jnp.int32, (_BLK, _BLK), 0)
           ).astype(jnp.float32)

    def excl_cumsum(oh, carry0):
        blocks = []
        carry = carry0
        for i in range(_NRB):
            blk = oh[i * _BLK:(i + 1) * _BLK, :]
            c = jnp.dot(tri, blk, preferred_element_type=jnp.float32) + carry
            blocks.append(c)
            carry = carry + jnp.sum(blk, axis=0, keepdims=True)
        return jnp.concatenate(blocks, axis=0), carry

    zero8 = jnp.zeros((1, _E), jnp.float32)
    c0, tot0 = excl_cumsum(oh0, zero8)
    c1, tot01 = excl_cumsum(oh1, tot0)

    rank0 = jnp.sum(jnp.where(onehot1, c0, 0.0), axis=-1, keepdims=True)
    rank1 = jnp.sum(jnp.where(onehot2, c1, 0.0), axis=-1, keepdims=True)

    counts = tot01.astype(jnp.int32)
    nb = (counts + (_BLK - 1)) // _BLK
    lane8 = jax.lax.broadcasted_iota(jnp.int32, (1, _E), 1)
    nb_s = [jnp.sum(jnp.where(lane8 == e, nb, 0)) for e in range(_E)]
    cb_s = []
    acc = nb_s[0]
    for e in range(_E):
        if e:
            acc = acc + nb_s[e]
        cb_s.append(acc)
    off_s = [(cb_s[e] - nb_s[e]) * _BLK for e in range(_E)]
    off = jnp.zeros((1, _E), jnp.int32)
    for e in range(_E):
        off = jnp.where(lane8 == e, off_s[e], off)

    offf = off.astype(jnp.float32)
    off0 = jnp.sum(jnp.where(onehot1, offf, 0.0), axis=-1, keepdims=True)
    off1 = jnp.sum(jnp.where(onehot2, offf, 0.0), axis=-1, keepdims=True)
    slot0 = (off0 + rank0).astype(jnp.int32)
    slot1 = (off1 + rank1).astype(jnp.int32)
    slots_ref[...] = (jnp.where(lane == 0, slot0, 0)
                      + jnp.where(lane == 1, slot1, 0))

    nact = cb_s[_E - 1]
    lane128 = jax.lax.broadcasted_iota(jnp.int32, (1, 128), 1)

    has = [(nb_s[e] > 0).astype(jnp.int32) for e in range(_E)]
    rank = []
    racc = jnp.zeros((), jnp.int32)
    for e in range(_E):
        rank.append(racc)
        racc = racc + has[e]
    nruns = racc

    def block_tables(brow):
        raw = jnp.zeros(brow.shape, jnp.int32)
        for e in range(_E):
            raw = raw + (brow >= cb_s[e]).astype(jnp.int32)
        return raw

    raw0 = block_tables(lane128)
    lastex = jnp.max(jnp.where(lane128 < nact, raw0, 0))
    be = jnp.minimum(raw0, lastex)
    xmap = jnp.minimum(lane128 - 32, nact - 1)

    bew_rid = jnp.minimum(block_tables(lane128 - 80), lastex)
    rid = jnp.zeros((1, 128), jnp.int32)
    for e in range(_E):
        rid = jnp.where(bew_rid == e, rank[e], rid)

    bew_st = jnp.minimum(block_tables(lane128 - 104), lastex)
    st = jnp.zeros((1, 128), jnp.int32)
    for e in range(_E):
        st = jnp.where(
            jnp.logical_and(bew_st == e,
                            (lane128 - 104) == cb_s[e] - nb_s[e]),
            1, st)
    st = jnp.where(lane128 - 104 < nact, st, 0)

    re = jnp.zeros((1, 128), jnp.int32)
    for e in range(_E):
        re = jnp.where(
            jnp.logical_and(lane128 == 64 + rank[e], has[e] > 0), e, re)

    meta = jnp.where(lane128 < _NBMAX, be, 0)
    meta = jnp.where((lane128 >= 32) & (lane128 < 32 + _NBMAX), xmap, meta)
    meta = jnp.where(lane128 == 56, nact, meta)
    meta = jnp.where((lane128 >= 64) & (lane128 < 64 + _E), re, meta)
    meta = jnp.where(lane128 == 72, nruns, meta)
    meta = jnp.where((lane128 >= 80) & (lane128 < 80 + _NBMAX), rid, meta)
    meta = jnp.where(lane128 >= 104, st, meta)
    meta_ref[...] = meta


def _routing(x, gate_w):
    return pl.pallas_call(
        _routing_body,
        out_shape=[
            jax.ShapeDtypeStruct((_S, _E), jnp.float32),
            jax.ShapeDtypeStruct((_S, 32), jnp.float32),
            jax.ShapeDtypeStruct((_S, _E), jnp.int32),
            jax.ShapeDtypeStruct((1, 128), jnp.int32),
        ],
    )(x, gate_w)


def _w_copies(w1_hbm, w3_hbm, w2_hbm, vw1, vw3, vw2, sem, e, slot):
    return (
        pltpu.make_async_copy(w1_hbm.at[e], vw1.at[slot], sem.at[slot]),
        pltpu.make_async_copy(w3_hbm.at[e], vw3.at[slot], sem.at[slot]),
        pltpu.make_async_copy(w2_hbm.at[e], vw2.at[slot], sem.at[slot]),
    )


def _gmm_body(meta_ref, xs_ref, w1_hbm, w3_hbm, w2_hbm, ds_ref,
              vw1, vw3, vw2, sem):
    b = pl.program_id(0)
    nact = meta_ref[0, 56]
    nruns = meta_ref[0, 72]
    rid = meta_ref[0, 80 + b]
    is_start = meta_ref[0, 104 + b]
    slot = jax.lax.rem(rid, 2)

    @pl.when(b == 0)
    def _():
        for c in _w_copies(w1_hbm, w3_hbm, w2_hbm, vw1, vw3, vw2, sem,
                           meta_ref[0, 64], 0):
            c.start()

        @pl.when(nruns > 1)
        def _():
            for c in _w_copies(w1_hbm, w3_hbm, w2_hbm, vw1, vw3, vw2, sem,
                               meta_ref[0, 65], 1):
                c.start()

    @pl.when(jnp.logical_and(b < nact, is_start == 1))
    def _():
        for c in _w_copies(w1_hbm, w3_hbm, w2_hbm, vw1, vw3, vw2, sem,
                           meta_ref[0, 64 + rid], slot):
            c.wait()

        @pl.when(jnp.logical_and(rid + 1 < nruns, rid >= 1))
        def _():
            for c in _w_copies(w1_hbm, w3_hbm, w2_hbm, vw1, vw3, vw2, sem,
                               meta_ref[0, 64 + rid + 1], 1 - slot):
                c.start()

    @pl.when(b < nact)
    def _():
        x = xs_ref[...]
        g = jnp.dot(x, vw1[slot], preferred_element_type=jnp.float32)
        u = jnp.dot(x, vw3[slot], preferred_element_type=jnp.float32)
        h = g * (1.0 / (1.0 + jnp.exp(-g))) * u
        ds_ref[...] = jnp.dot(h, vw2[slot], preferred_element_type=jnp.float32)


def _gmm(meta, xs, w1, w3, w2):
    grid_spec = pltpu.PrefetchScalarGridSpec(
        num_scalar_prefetch=1,
        grid=(_NBMAX,),
        in_specs=[
            pl.BlockSpec((_BLK, _D), lambda b, m: (m[0, 32 + b], 0)),
            pl.BlockSpec(memory_space=pl.ANY),
            pl.BlockSpec(memory_space=pl.ANY),
            pl.BlockSpec(memory_space=pl.ANY),
        ],
        out_specs=pl.BlockSpec((_BLK, _D), lambda b, m: (b, 0)),
        scratch_shapes=[
            pltpu.VMEM((2, _D, _F), jnp.float32),
            pltpu.VMEM((2, _D, _F), jnp.float32),
            pltpu.VMEM((2, _F, _D), jnp.float32),
            pltpu.SemaphoreType.DMA((2,)),
        ],
    )
    return pl.pallas_call(
        _gmm_body,
        grid_spec=grid_spec,
        out_shape=jax.ShapeDtypeStruct((_NSLOT, _D), jnp.float32),
        compiler_params=pltpu.CompilerParams(
            dimension_semantics=("arbitrary",),
            vmem_limit_bytes=100 * 1024 * 1024,
        ),
    )(meta, xs, w1, w3, w2)


def _dispatch(x, s0, s1):
    mesh = plsc.VectorSubcoreMesh(core_axis_name="c", subcore_axis_name="s")

    @functools.partial(
        pl.kernel, mesh=mesh,
        out_type=jax.ShapeDtypeStruct((_NSLOT, _D), jnp.float32),
        scratch_types=[
            pltpu.VMEM((_TPW, _D), jnp.float32),
            pltpu.VMEM((_TPW,), jnp.int32),
            pltpu.VMEM((_TPW,), jnp.int32),
            pltpu.SemaphoreType.DMA,
            pltpu.SemaphoreType.DMA,
        ],
    )
    def k(x_hbm, s0_hbm, s1_hbm, out_hbm, xbuf, i0, i1, sem0, sem1):
        wid = lax.axis_index("s") * 2 + lax.axis_index("c")
        base = wid * _TPW
        pltpu.sync_copy(x_hbm.at[pl.ds(base, _TPW)], xbuf)
        pltpu.sync_copy(s0_hbm.at[wid], i0)
        pltpu.sync_copy(s1_hbm.at[wid], i1)
        c0 = pltpu.async_copy(xbuf, out_hbm.at[i0], sem0)
        c1 = pltpu.async_copy(xbuf, out_hbm.at[i1], sem1)
        c0.wait()
        c1.wait()

    return k(x, s0, s1)


def _combine(ds, s0, s1, mult):
    mesh = plsc.VectorSubcoreMesh(core_axis_name="c", subcore_axis_name="s")

    @functools.partial(
        pl.kernel, mesh=mesh,
        out_type=jax.ShapeDtypeStruct((_S, _D), jnp.float32),
        scratch_types=[
            pltpu.VMEM((_TPW, _D), jnp.float32),
            pltpu.VMEM((_TPW, _D), jnp.float32),
            pltpu.VMEM((_TPW, 32), jnp.float32),
            pltpu.VMEM((_TPW,), jnp.int32),
            pltpu.VMEM((_TPW,), jnp.int32),
            pltpu.SemaphoreType.DMA,
            pltpu.SemaphoreType.DMA,
        ],
    )
    def k(ds_hbm, s0_hbm, s1_hbm, m_hbm, out_hbm,
          buf0, buf1, mbuf, i0, i1, sem0, sem1):
        wid = lax.axis_index("s") * 2 + lax.axis_index("c")
        base = wid * _TPW
        pltpu.sync_copy(s0_hbm.at[wid], i0)
        pltpu.sync_copy(s1_hbm.at[wid], i1)
        pltpu.sync_copy(m_hbm.at[pl.ds(base, _TPW)], mbuf)
        c0 = pltpu.async_copy(ds_hbm.at[i0], buf0, sem0)
        c1 = pltpu.async_copy(ds_hbm.at[i1], buf1, sem1)
        c0.wait()
        c1.wait()

        def row(j, _):
            w0 = mbuf[j, pl.ds(0, 16)]
            w1 = mbuf[j, pl.ds(16, 16)]
            for c in range(_D // 16):
                cs = pl.ds(c * 16, 16)
                buf0[j, cs] = buf0[j, cs] * w0 + buf1[j, cs] * w1
            return ()

        lax.fori_loop(0, _TPW, row, ())
        pltpu.sync_copy(buf0, out_hbm.at[pl.ds(base, _TPW)])

    return k(ds, s0, s1, mult)


def kernel(hidden_states, gate_w, w1, w2, w3):
    x = hidden_states.reshape(-1, _D)
    logits, mult, slots, meta = _routing(x, gate_w)
    s0 = slots[:, 0].reshape(_NW, _TPW)
    s1 = slots[:, 1].reshape(_NW, _TPW)
    xs = _dispatch(x, s0, s1)
    ds = _gmm(meta, xs, w1, w3, w2)
    out = _combine(ds, s0, s1, mult)
    return (out.reshape(hidden_states.shape),
            logits.reshape(_B, _S, _E))


# speedup vs baseline: 1.6361x; 1.0055x over previous
import functools

import jax
import jax.numpy as jnp
from jax import lax
from jax.experimental import pallas as pl
from jax.experimental.pallas import tpu as pltpu
from jax.experimental.pallas import tpu_sc as plsc

_B, _S, _D = 1, 2048, 768
_F = 3072
_E = 8
_JITTER = 0.01

_BLK = 256
_NBMAX = 24
_NSLOT = _NBMAX * _BLK
_NW = 32
_TPW = _S // _NW

_NRB = _S // _BLK


def _routing_body(x_ref, gw_ref, logits_ref, mult_ref, slots_ref, meta_ref):
    x = x_ref[...]
    gw = gw_ref[...]
    scores = jnp.dot(x, gw, preferred_element_type=jnp.float32)
    logits_ref[...] = scores

    neg = jnp.float32(-jnp.inf)
    lane = jax.lax.broadcasted_iota(jnp.int32, (_S, _E), 1)

    max_vals = jnp.max(scores, axis=-1, keepdims=True)
    max_ind = jnp.min(jnp.where(scores == max_vals, lane, _E),
                      axis=-1, keepdims=True)
    factor = jnp.maximum(jnp.abs(scores), max_vals)
    mask1 = (max_vals - scores) / factor > 2 * _JITTER
    mg = jnp.where(mask1, neg, scores)
    m = jnp.max(mg, axis=-1, keepdims=True)
    p = jnp.exp(mg - m)
    p = p / jnp.sum(p, axis=-1, keepdims=True)
    onehot1 = lane == max_ind
    mult1 = jnp.sum(jnp.where(onehot1, p, 0.0), axis=-1, keepdims=True)

    masked_scores = jnp.where(onehot1, neg, scores)
    max_vals2 = jnp.max(masked_scores, axis=-1, keepdims=True)
    max_ind2 = jnp.min(jnp.where(masked_scores == max_vals2, lane, _E),
                       axis=-1, keepdims=True)
    factor2 = jnp.maximum(jnp.abs(scores), max_vals2)
    mask2 = (max_vals2 - scores) / factor2 > 2 * _JITTER
    mg2 = jnp.where(mask2, neg, masked_scores)
    m2 = jnp.max(mg2, axis=-1, keepdims=True)
    p2 = jnp.exp(mg2 - m2)
    p2 = p2 / jnp.sum(p2, axis=-1, keepdims=True)
    onehot2 = lane == max_ind2
    mult2 = jnp.sum(jnp.where(onehot2, p2, 0.0), axis=-1, keepdims=True)

    mult_ref[...] = jnp.concatenate(
        [jnp.broadcast_to(mult1, (_S, 16)),
         jnp.broadcast_to(mult2, (_S, 16))], axis=-1)

    oh0 = onehot1.astype(jnp.float32)
    oh1 = onehot2.astype(jnp.float32)
    tri = (jax.lax.broadcasted_iota(jnp.int32, (_BLK, _BLK), 1)
           < jax.lax.broadcasted_iota(jnp.int32, (_BLK, _BLK), 0)
           ).astype(jnp.float32)

    def excl_cumsum(oh, carry0):
        blocks = []
        carry = carry0
        for i in range(_NRB):
            blk = oh[i * _BLK:(i + 1) * _BLK, :]
            c = jnp.dot(tri, blk, preferred_element_type=jnp.float32) + carry
            blocks.append(c)
            carry = carry + jnp.sum(blk, axis=0, keepdims=True)
        return jnp.concatenate(blocks, axis=0), carry

    zero8 = jnp.zeros((1, _E), jnp.float32)
    c0, tot0 = excl_cumsum(oh0, zero8)
    c1, tot01 = excl_cumsum(oh1, tot0)

    rank0 = jnp.sum(jnp.where(onehot1, c0, 0.0), axis=-1, keepdims=True)
    rank1 = jnp.sum(jnp.where(onehot2, c1, 0.0), axis=-1, keepdims=True)

    counts = tot01.astype(jnp.int32)
    nb = (counts + (_BLK - 1)) // _BLK
    lane8 = jax.lax.broadcasted_iota(jnp.int32, (1, _E), 1)
    nb_s = [jnp.sum(jnp.where(lane8 == e, nb, 0)) for e in range(_E)]
    cb_s = []
    acc = nb_s[0]
    for e in range(_E):
        if e:
            acc = acc + nb_s[e]
        cb_s.append(acc)
    off_s = [(cb_s[e] - nb_s[e]) * _BLK for e in range(_E)]
    off = jnp.zeros((1, _E), jnp.int32)
    for e in range(_E):
        off = jnp.where(lane8 == e, off_s[e], off)

    offf = off.astype(jnp.float32)
    off0 = jnp.sum(jnp.where(onehot1, offf, 0.0), axis=-1, keepdims=True)
    off1 = jnp.sum(jnp.where(onehot2, offf, 0.0), axis=-1, keepdims=True)
    slot0 = (off0 + rank0).astype(jnp.int32)
    slot1 = (off1 + rank1).astype(jnp.int32)
    slots_ref[...] = (jnp.where(lane == 0, slot0, 0)
                      + jnp.where(lane == 1, slot1, 0))

    nact = cb_s[_E - 1]
    lane128 = jax.lax.broadcasted_iota(jnp.int32, (1, 128), 1)

    has = [(nb_s[e] > 0).astype(jnp.int32) for e in range(_E)]
    rank = []
    racc = jnp.zeros((), jnp.int32)
    for e in range(_E):
        rank.append(racc)
        racc = racc + has[e]
    nruns = racc

    def block_tables(brow):
        raw = jnp.zeros(brow.shape, jnp.int32)
        for e in range(_E):
            raw = raw + (brow >= cb_s[e]).astype(jnp.int32)
        return raw

    raw0 = block_tables(lane128)
    lastex = jnp.max(jnp.where(lane128 < nact, raw0, 0))
    be = jnp.minimum(raw0, lastex)
    xmap = jnp.minimum(lane128 - 32, nact - 1)

    bew_rid = jnp.minimum(block_tables(lane128 - 80), lastex)
    rid = jnp.zeros((1, 128), jnp.int32)
    for e in range(_E):
        rid = jnp.where(bew_rid == e, rank[e], rid)

    bew_st = jnp.minimum(block_tables(lane128 - 104), lastex)
    st = jnp.zeros((1, 128), jnp.int32)
    for e in range(_E):
        st = jnp.where(
            jnp.logical_and(bew_st == e,
                            (lane128 - 104) == cb_s[e] - nb_s[e]),
            1, st)
    st = jnp.where(lane128 - 104 < nact, st, 0)

    re = jnp.zeros((1, 128), jnp.int32)
    for e in range(_E):
        re = jnp.where(
            jnp.logical_and(lane128 == 64 + rank[e], has[e] > 0), e, re)

    meta = jnp.where(lane128 < _NBMAX, be, 0)
    meta = jnp.where((lane128 >= 32) & (lane128 < 32 + _NBMAX), xmap, meta)
    meta = jnp.where(lane128 == 56, nact, meta)
    meta = jnp.where((lane128 >= 64) & (lane128 < 64 + _E), re, meta)
    meta = jnp.where(lane128 == 72, nruns, meta)
    meta = jnp.where((lane128 >= 80) & (lane128 < 80 + _NBMAX), rid, meta)
    meta = jnp.where(lane128 >= 104, st, meta)
    meta_ref[...] = meta


def _routing(x, gate_w):
    return pl.pallas_call(
        _routing_body,
        out_shape=[
            jax.ShapeDtypeStruct((_S, _E), jnp.float32),
            jax.ShapeDtypeStruct((_S, 32), jnp.float32),
            jax.ShapeDtypeStruct((_S, _E), jnp.int32),
            jax.ShapeDtypeStruct((1, 128), jnp.int32),
        ],
    )(x, gate_w)


def _w_copies(w1_hbm, w3_hbm, w2_hbm, vw1, vw3, vw2, sem, e, slot):
    return (
        pltpu.make_async_copy(w1_hbm.at[e], vw1.at[slot], sem.at[slot]),
        pltpu.make_async_copy(w3_hbm.at[e], vw3.at[slot], sem.at[slot]),
        pltpu.make_async_copy(w2_hbm.at[e], vw2.at[slot], sem.at[slot]),
    )


def _gmm_body(meta_ref, xs_ref, w1_hbm, w3_hbm, w2_hbm, ds_ref,
              vw1, vw3, vw2, sem):
    b = pl.program_id(0)
    nact = meta_ref[0, 56]
    nruns = meta_ref[0, 72]
    rid = meta_ref[0, 80 + b]
    is_start = meta_ref[0, 104 + b]
    slot = jax.lax.rem(rid, 2)

    @pl.when(b == 0)
    def _():
        for c in _w_copies(w1_hbm, w3_hbm, w2_hbm, vw1, vw3, vw2, sem,
                           meta_ref[0, 64], 0):
            c.start()

        @pl.when(nruns > 1)
        def _():
            for c in _w_copies(w1_hbm, w3_hbm, w2_hbm, vw1, vw3, vw2, sem,
                               meta_ref[0, 65], 1):
                c.start()

    @pl.when(jnp.logical_and(b < nact, is_start == 1))
    def _():
        for c in _w_copies(w1_hbm, w3_hbm, w2_hbm, vw1, vw3, vw2, sem,
                           meta_ref[0, 64 + rid], slot):
            c.wait()

        @pl.when(jnp.logical_and(rid + 1 < nruns, rid >= 1))
        def _():
            for c in _w_copies(w1_hbm, w3_hbm, w2_hbm, vw1, vw3, vw2, sem,
                               meta_ref[0, 64 + rid + 1], 1 - slot):
                c.start()

    @pl.when(b < nact)
    def _():
        x = xs_ref[...]
        g = jnp.dot(x, vw1[slot], preferred_element_type=jnp.float32)
        u = jnp.dot(x, vw3[slot], preferred_element_type=jnp.float32)
        h = g * (1.0 / (1.0 + jnp.exp(-g))) * u
        ds_ref[...] = jnp.dot(h, vw2[slot], preferred_element_type=jnp.float32)


def _gmm(meta, xs, w1, w3, w2):
    grid_spec = pltpu.PrefetchScalarGridSpec(
        num_scalar_prefetch=1,
        grid=(_NBMAX,),
        in_specs=[
            pl.BlockSpec((_BLK, _D), lambda b, m: (m[0, 32 + b], 0)),
            pl.BlockSpec(memory_space=pl.ANY),
            pl.BlockSpec(memory_space=pl.ANY),
            pl.BlockSpec(memory_space=pl.ANY),
        ],
        out_specs=pl.BlockSpec((_BLK, _D), lambda b, m: (b, 0)),
        scratch_shapes=[
            pltpu.VMEM((2, _D, _F), jnp.float32),
            pltpu.VMEM((2, _D, _F), jnp.float32),
            pltpu.VMEM((2, _F, _D), jnp.float32),
            pltpu.SemaphoreType.DMA((2,)),
        ],
    )
    return pl.pallas_call(
        _gmm_body,
        grid_spec=grid_spec,
        out_shape=jax.ShapeDtypeStruct((_NSLOT, _D), jnp.float32),
        compiler_params=pltpu.CompilerParams(
            dimension_semantics=("arbitrary",),
            vmem_limit_bytes=100 * 1024 * 1024,
        ),
    )(meta, xs, w1, w3, w2)


def _dispatch(x, s0, s1):
    mesh = plsc.VectorSubcoreMesh(core_axis_name="c", subcore_axis_name="s")

    @functools.partial(
        pl.kernel, mesh=mesh,
        out_type=jax.ShapeDtypeStruct((_NSLOT, _D), jnp.float32),
        scratch_types=[
            pltpu.VMEM((_TPW, _D), jnp.float32),
            pltpu.VMEM((2, _TPW // 2), jnp.int32),
            pltpu.VMEM((2, _TPW // 2), jnp.int32),
            pltpu.SemaphoreType.DMA,
            pltpu.SemaphoreType.DMA,
            pltpu.SemaphoreType.DMA,
        ],
    )
    def k(x_hbm, s0_hbm, s1_hbm, out_hbm, xbuf, i0, i1, sem0, sem1, semx):
        wid = lax.axis_index("s") * 2 + lax.axis_index("c")
        base = wid * _TPW
        h = _TPW // 2
        cxa = pltpu.async_copy(x_hbm.at[pl.ds(base, h)],
                               xbuf.at[pl.ds(0, h)], semx)
        cxb = pltpu.async_copy(x_hbm.at[pl.ds(base + h, h)],
                               xbuf.at[pl.ds(h, h)], semx)
        pltpu.sync_copy(s0_hbm.at[wid], i0)
        pltpu.sync_copy(s1_hbm.at[wid], i1)
        cxa.wait()
        c0a = pltpu.async_copy(xbuf.at[pl.ds(0, h)],
                               out_hbm.at[i0.at[0]], sem0)
        c1a = pltpu.async_copy(xbuf.at[pl.ds(0, h)],
                               out_hbm.at[i1.at[0]], sem1)
        cxb.wait()
        c0b = pltpu.async_copy(xbuf.at[pl.ds(h, h)],
                               out_hbm.at[i0.at[1]], sem0)
        c1b = pltpu.async_copy(xbuf.at[pl.ds(h, h)],
                               out_hbm.at[i1.at[1]], sem1)
        c0a.wait()
        c1a.wait()
        c0b.wait()
        c1b.wait()

    return k(x, s0, s1)


def _combine(ds, s0, s1, mult):
    mesh = plsc.VectorSubcoreMesh(core_axis_name="c", subcore_axis_name="s")

    @functools.partial(
        pl.kernel, mesh=mesh,
        out_type=jax.ShapeDtypeStruct((_S, _D), jnp.float32),
        scratch_types=[
            pltpu.VMEM((_TPW, _D), jnp.float32),
            pltpu.VMEM((_TPW, _D), jnp.float32),
            pltpu.VMEM((_TPW, 32), jnp.float32),
            pltpu.VMEM((_TPW,), jnp.int32),
            pltpu.VMEM((_TPW,), jnp.int32),
            pltpu.SemaphoreType.DMA,
            pltpu.SemaphoreType.DMA,
        ],
    )
    def k(ds_hbm, s0_hbm, s1_hbm, m_hbm, out_hbm,
          buf0, buf1, mbuf, i0, i1, sem0, sem1):
        wid = lax.axis_index("s") * 2 + lax.axis_index("c")
        base = wid * _TPW
        pltpu.sync_copy(s0_hbm.at[wid], i0)
        pltpu.sync_copy(s1_hbm.at[wid], i1)
        pltpu.sync_copy(m_hbm.at[pl.ds(base, _TPW)], mbuf)
        c0 = pltpu.async_copy(ds_hbm.at[i0], buf0, sem0)
        c1 = pltpu.async_copy(ds_hbm.at[i1], buf1, sem1)
        c0.wait()
        c1.wait()

        def row(j, _):
            w0 = mbuf[j, pl.ds(0, 16)]
            w1 = mbuf[j, pl.ds(16, 16)]
            for c in range(_D // 16):
                cs = pl.ds(c * 16, 16)
                buf0[j, cs] = buf0[j, cs] * w0 + buf1[j, cs] * w1
            return ()

        lax.fori_loop(0, _TPW, row, ())
        pltpu.sync_copy(buf0, out_hbm.at[pl.ds(base, _TPW)])

    return k(ds, s0, s1, mult)


def kernel(hidden_states, gate_w, w1, w2, w3):
    x = hidden_states.reshape(-1, _D)
    logits, mult, slots, meta = _routing(x, gate_w)
    s0 = slots[:, 0].reshape(_NW, _TPW)
    s1 = slots[:, 1].reshape(_NW, _TPW)
    xs = _dispatch(x,
                   s0.reshape(_NW, 2, _TPW // 2),
                   s1.reshape(_NW, 2, _TPW // 2))
    ds = _gmm(meta, xs, w1, w3, w2)
    out = _combine(ds, s0, s1, mult)
    return (out.reshape(hidden_states.shape),
            logits.reshape(_B, _S, _E))


# speedup vs baseline: 1.6408x; 1.0029x over previous
import functools

import jax
import jax.numpy as jnp
from jax import lax
from jax.experimental import pallas as pl
from jax.experimental.pallas import tpu as pltpu
from jax.experimental.pallas import tpu_sc as plsc

_B, _S, _D = 1, 2048, 768
_F = 3072
_E = 8
_JITTER = 0.01

_BLK = 256
_NBMAX = 24
_NSLOT = _NBMAX * _BLK
_NW = 32
_TPW = _S // _NW

_NRB = _S // _BLK


def _routing_body(x_ref, gw_ref, logits_ref, mult_ref, slots_ref, meta_ref):
    x = x_ref[...]
    gw = gw_ref[...]
    scores = jnp.dot(x, gw, preferred_element_type=jnp.float32)
    logits_ref[...] = scores

    neg = jnp.float32(-jnp.inf)
    lane = jax.lax.broadcasted_iota(jnp.int32, (_S, _E), 1)

    max_vals = jnp.max(scores, axis=-1, keepdims=True)
    max_ind = jnp.min(jnp.where(scores == max_vals, lane, _E),
                      axis=-1, keepdims=True)
    factor = jnp.maximum(jnp.abs(scores), max_vals)
    mask1 = (max_vals - scores) / factor > 2 * _JITTER
    mg = jnp.where(mask1, neg, scores)
    m = jnp.max(mg, axis=-1, keepdims=True)
    p = jnp.exp(mg - m)
    p = p / jnp.sum(p, axis=-1, keepdims=True)
    onehot1 = lane == max_ind
    mult1 = jnp.sum(jnp.where(onehot1, p, 0.0), axis=-1, keepdims=True)

    masked_scores = jnp.where(onehot1, neg, scores)
    max_vals2 = jnp.max(masked_scores, axis=-1, keepdims=True)
    max_ind2 = jnp.min(jnp.where(masked_scores == max_vals2, lane, _E),
                       axis=-1, keepdims=True)
    factor2 = jnp.maximum(jnp.abs(scores), max_vals2)
    mask2 = (max_vals2 - scores) / factor2 > 2 * _JITTER
    mg2 = jnp.where(mask2, neg, masked_scores)
    m2 = jnp.max(mg2, axis=-1, keepdims=True)
    p2 = jnp.exp(mg2 - m2)
    p2 = p2 / jnp.sum(p2, axis=-1, keepdims=True)
    onehot2 = lane == max_ind2
    mult2 = jnp.sum(jnp.where(onehot2, p2, 0.0), axis=-1, keepdims=True)

    mult_ref[...] = jnp.concatenate(
        [jnp.broadcast_to(mult1, (_S, 16)),
         jnp.broadcast_to(mult2, (_S, 16))], axis=-1)

    oh0 = onehot1.astype(jnp.float32)
    oh1 = onehot2.astype(jnp.float32)
    tri = (jax.lax.broadcasted_iota(jnp.int32, (_BLK, _BLK), 1)
           < jax.lax.broadcasted_iota(jnp.int32, (_BLK, _BLK), 0)
           ).astype(jnp.float32)

    def excl_cumsum(oh, carry0):
        blocks = []
        carry = carry0
        for i in range(_NRB):
            blk = oh[i * _BLK:(i + 1) * _BLK, :]
            c = jnp.dot(tri, blk, preferred_element_type=jnp.float32) + carry
            blocks.append(c)
            carry = carry + jnp.sum(blk, axis=0, keepdims=True)
        return jnp.concatenate(blocks, axis=0), carry

    zero8 = jnp.zeros((1, _E), jnp.float32)
    c0, tot0 = excl_cumsum(oh0, zero8)
    c1, tot01 = excl_cumsum(oh1, tot0)

    rank0 = jnp.sum(jnp.where(onehot1, c0, 0.0), axis=-1, keepdims=True)
    rank1 = jnp.sum(jnp.where(onehot2, c1, 0.0), axis=-1, keepdims=True)

    counts = tot01.astype(jnp.int32)
    nb = (counts + (_BLK - 1)) // _BLK
    lane8 = jax.lax.broadcasted_iota(jnp.int32, (1, _E), 1)
    nb_s = [jnp.sum(jnp.where(lane8 == e, nb, 0)) for e in range(_E)]
    cb_s = []
    acc = nb_s[0]
    for e in range(_E):
        if e:
            acc = acc + nb_s[e]
        cb_s.append(acc)
    off_s = [(cb_s[e] - nb_s[e]) * _BLK for e in range(_E)]
    off = jnp.zeros((1, _E), jnp.int32)
    for e in range(_E):
        off = jnp.where(lane8 == e, off_s[e], off)

    offf = off.astype(jnp.float32)
    off0 = jnp.sum(jnp.where(onehot1, offf, 0.0), axis=-1, keepdims=True)
    off1 = jnp.sum(jnp.where(onehot2, offf, 0.0), axis=-1, keepdims=True)
    slot0 = (off0 + rank0).astype(jnp.int32)
    slot1 = (off1 + rank1).astype(jnp.int32)
    slots_ref[...] = (jnp.where(lane == 0, slot0, 0)
                      + jnp.where(lane == 1, slot1, 0))

    nact = cb_s[_E - 1]
    lane128 = jax.lax.broadcasted_iota(jnp.int32, (1, 128), 1)

    has = [(nb_s[e] > 0).astype(jnp.int32) for e in range(_E)]
    rank = []
    racc = jnp.zeros((), jnp.int32)
    for e in range(_E):
        rank.append(racc)
        racc = racc + has[e]
    nruns = racc

    def block_tables(brow):
        raw = jnp.zeros(brow.shape, jnp.int32)
        for e in range(_E):
            raw = raw + (brow >= cb_s[e]).astype(jnp.int32)
        return raw

    raw0 = block_tables(lane128)
    lastex = jnp.max(jnp.where(lane128 < nact, raw0, 0))
    be = jnp.minimum(raw0, lastex)
    xmap = jnp.minimum(lane128 - 32, nact - 1)

    bew_rid = jnp.minimum(block_tables(lane128 - 80), lastex)
    rid = jnp.zeros((1, 128), jnp.int32)
    for e in range(_E):
        rid = jnp.where(bew_rid == e, rank[e], rid)

    bew_st = jnp.minimum(block_tables(lane128 - 104), lastex)
    st = jnp.zeros((1, 128), jnp.int32)
    for e in range(_E):
        st = jnp.where(
            jnp.logical_and(bew_st == e,
                            (lane128 - 104) == cb_s[e] - nb_s[e]),
            1, st)
    st = jnp.where(lane128 - 104 < nact, st, 0)

    re = jnp.zeros((1, 128), jnp.int32)
    for e in range(_E):
        re = jnp.where(
            jnp.logical_and(lane128 == 64 + rank[e], has[e] > 0), e, re)

    meta = jnp.where(lane128 < _NBMAX, be, 0)
    meta = jnp.where((lane128 >= 32) & (lane128 < 32 + _NBMAX), xmap, meta)
    meta = jnp.where(lane128 == 56, nact, meta)
    meta = jnp.where((lane128 >= 64) & (lane128 < 64 + _E), re, meta)
    meta = jnp.where(lane128 == 72, nruns, meta)
    meta = jnp.where((lane128 >= 80) & (lane128 < 80 + _NBMAX), rid, meta)
    meta = jnp.where(lane128 >= 104, st, meta)
    meta_ref[...] = meta


def _routing(x, gate_w):
    return pl.pallas_call(
        _routing_body,
        out_shape=[
            jax.ShapeDtypeStruct((_S, _E), jnp.float32),
            jax.ShapeDtypeStruct((_S, 32), jnp.float32),
            jax.ShapeDtypeStruct((_S, _E), jnp.int32),
            jax.ShapeDtypeStruct((1, 128), jnp.int32),
        ],
    )(x, gate_w)


def _w_copies(w1_hbm, w3_hbm, w2_hbm, vw1, vw3, vw2, sem, e, slot):
    return (
        pltpu.make_async_copy(w1_hbm.at[e], vw1.at[slot], sem.at[slot]),
        pltpu.make_async_copy(w3_hbm.at[e], vw3.at[slot], sem.at[slot]),
        pltpu.make_async_copy(w2_hbm.at[e], vw2.at[slot], sem.at[slot]),
    )


def _gmm_body(meta_ref, xs_ref, w1_hbm, w3_hbm, w2_hbm, ds_ref,
              vw1, vw3, vw2, sem):
    b = pl.program_id(0)
    nact = meta_ref[0, 56]
    nruns = meta_ref[0, 72]
    rid = meta_ref[0, 80 + b]
    is_start = meta_ref[0, 104 + b]
    slot = jax.lax.rem(rid, 2)

    @pl.when(b == 0)
    def _():
        for c in _w_copies(w1_hbm, w3_hbm, w2_hbm, vw1, vw3, vw2, sem,
                           meta_ref[0, 64], 0):
            c.start()

        @pl.when(nruns > 1)
        def _():
            for c in _w_copies(w1_hbm, w3_hbm, w2_hbm, vw1, vw3, vw2, sem,
                               meta_ref[0, 65], 1):
                c.start()

    @pl.when(jnp.logical_and(b < nact, is_start == 1))
    def _():
        for c in _w_copies(w1_hbm, w3_hbm, w2_hbm, vw1, vw3, vw2, sem,
                           meta_ref[0, 64 + rid], slot):
            c.wait()

        @pl.when(jnp.logical_and(rid + 1 < nruns, rid >= 1))
        def _():
            for c in _w_copies(w1_hbm, w3_hbm, w2_hbm, vw1, vw3, vw2, sem,
                               meta_ref[0, 64 + rid + 1], 1 - slot):
                c.start()

    @pl.when(b < nact)
    def _():
        x = xs_ref[...]
        g = jnp.dot(x, vw1[slot], preferred_element_type=jnp.float32)
        u = jnp.dot(x, vw3[slot], preferred_element_type=jnp.float32)
        h = g * (1.0 / (1.0 + jnp.exp(-g))) * u
        ds_ref[...] = jnp.dot(h, vw2[slot], preferred_element_type=jnp.float32)


def _gmm(meta, xs, w1, w3, w2):
    grid_spec = pltpu.PrefetchScalarGridSpec(
        num_scalar_prefetch=1,
        grid=(_NBMAX,),
        in_specs=[
            pl.BlockSpec((_BLK, _D), lambda b, m: (m[0, 32 + b], 0)),
            pl.BlockSpec(memory_space=pl.ANY),
            pl.BlockSpec(memory_space=pl.ANY),
            pl.BlockSpec(memory_space=pl.ANY),
        ],
        out_specs=pl.BlockSpec((_BLK, _D), lambda b, m: (b, 0)),
        scratch_shapes=[
            pltpu.VMEM((2, _D, _F), jnp.float32),
            pltpu.VMEM((2, _D, _F), jnp.float32),
            pltpu.VMEM((2, _F, _D), jnp.float32),
            pltpu.SemaphoreType.DMA((2,)),
        ],
    )
    return pl.pallas_call(
        _gmm_body,
        grid_spec=grid_spec,
        out_shape=jax.ShapeDtypeStruct((_NSLOT, _D), jnp.float32),
        compiler_params=pltpu.CompilerParams(
            dimension_semantics=("arbitrary",),
            vmem_limit_bytes=100 * 1024 * 1024,
        ),
    )(meta, xs, w1, w3, w2)


def _dispatch(x, s0, s1):
    mesh = plsc.VectorSubcoreMesh(core_axis_name="c", subcore_axis_name="s")

    @functools.partial(
        pl.kernel, mesh=mesh,
        out_type=jax.ShapeDtypeStruct((_NSLOT, _D), jnp.float32),
        scratch_types=[
            pltpu.VMEM((_TPW, _D), jnp.float32),
            pltpu.VMEM((2, _TPW // 2), jnp.int32),
            pltpu.VMEM((2, _TPW // 2), jnp.int32),
            pltpu.SemaphoreType.DMA,
            pltpu.SemaphoreType.DMA,
            pltpu.SemaphoreType.DMA,
        ],
    )
    def k(x_hbm, s0_hbm, s1_hbm, out_hbm, xbuf, i0, i1, sem0, sem1, semx):
        wid = lax.axis_index("s") * 2 + lax.axis_index("c")
        base = wid * _TPW
        h = _TPW // 2
        cxa = pltpu.async_copy(x_hbm.at[pl.ds(base, h)],
                               xbuf.at[pl.ds(0, h)], semx)
        cxb = pltpu.async_copy(x_hbm.at[pl.ds(base + h, h)],
                               xbuf.at[pl.ds(h, h)], semx)
        pltpu.sync_copy(s0_hbm.at[wid], i0)
        pltpu.sync_copy(s1_hbm.at[wid], i1)
        cxa.wait()
        c0a = pltpu.async_copy(xbuf.at[pl.ds(0, h)],
                               out_hbm.at[i0.at[0]], sem0)
        c1a = pltpu.async_copy(xbuf.at[pl.ds(0, h)],
                               out_hbm.at[i1.at[0]], sem1)
        cxb.wait()
        c0b = pltpu.async_copy(xbuf.at[pl.ds(h, h)],
                               out_hbm.at[i0.at[1]], sem0)
        c1b = pltpu.async_copy(xbuf.at[pl.ds(h, h)],
                               out_hbm.at[i1.at[1]], sem1)
        c0a.wait()
        c1a.wait()
        c0b.wait()
        c1b.wait()

    return k(x, s0, s1)


def _combine(ds, s0, s1, mult):
    mesh = plsc.VectorSubcoreMesh(core_axis_name="c", subcore_axis_name="s")

    @functools.partial(
        pl.kernel, mesh=mesh,
        out_type=jax.ShapeDtypeStruct((_S, _D), jnp.float32),
        scratch_types=[
            pltpu.VMEM((_TPW, _D), jnp.float32),
            pltpu.VMEM((_TPW, _D), jnp.float32),
            pltpu.VMEM((_TPW, 32), jnp.float32),
            pltpu.VMEM((2, _TPW // 2), jnp.int32),
            pltpu.VMEM((2, _TPW // 2), jnp.int32),
            pltpu.SemaphoreType.DMA,
            pltpu.SemaphoreType.DMA,
        ],
    )
    def k(ds_hbm, s0_hbm, s1_hbm, m_hbm, out_hbm,
          buf0, buf1, mbuf, i0, i1, semA, semB):
        wid = lax.axis_index("s") * 2 + lax.axis_index("c")
        base = wid * _TPW
        h = _TPW // 2
        pltpu.sync_copy(s0_hbm.at[wid], i0)
        pltpu.sync_copy(s1_hbm.at[wid], i1)
        pltpu.sync_copy(m_hbm.at[pl.ds(base, _TPW)], mbuf)
        c0a = pltpu.async_copy(ds_hbm.at[i0.at[0]], buf0.at[pl.ds(0, h)],
                               semA)
        c1a = pltpu.async_copy(ds_hbm.at[i1.at[0]], buf1.at[pl.ds(0, h)],
                               semA)
        c0b = pltpu.async_copy(ds_hbm.at[i0.at[1]], buf0.at[pl.ds(h, h)],
                               semB)
        c1b = pltpu.async_copy(ds_hbm.at[i1.at[1]], buf1.at[pl.ds(h, h)],
                               semB)

        def row(j, _):
            w0 = mbuf[j, pl.ds(0, 16)]
            w1 = mbuf[j, pl.ds(16, 16)]
            for c in range(_D // 16):
                cs = pl.ds(c * 16, 16)
                buf0[j, cs] = buf0[j, cs] * w0 + buf1[j, cs] * w1
            return ()

        c0a.wait()
        c1a.wait()
        lax.fori_loop(0, h, row, ())
        pltpu.sync_copy(buf0.at[pl.ds(0, h)],
                        out_hbm.at[pl.ds(base, h)])
        c0b.wait()
        c1b.wait()
        lax.fori_loop(h, _TPW, row, ())
        pltpu.sync_copy(buf0.at[pl.ds(h, h)],
                        out_hbm.at[pl.ds(base + h, h)])

    return k(ds, s0, s1, mult)


def kernel(hidden_states, gate_w, w1, w2, w3):
    x = hidden_states.reshape(-1, _D)
    logits, mult, slots, meta = _routing(x, gate_w)
    s0 = slots[:, 0].reshape(_NW, 2, _TPW // 2)
    s1 = slots[:, 1].reshape(_NW, 2, _TPW // 2)
    xs = _dispatch(x, s0, s1)
    ds = _gmm(meta, xs, w1, w3, w2)
    out = _combine(ds, s0, s1, mult)
    return (out.reshape(hidden_states.shape),
            logits.reshape(_B, _S, _E))
